# Initial kernel scaffold; baseline (speedup 1.0000x reference)
import functools

import jax
import jax.numpy as jnp
from jax import lax
from jax.experimental import pallas as pl
from jax.experimental.pallas import tpu as pltpu
from jax.experimental.pallas import tpu_sc as plsc

N = 10000
E = 320000
C = 128

NP = 10240
NT = 16
EW = E // NT
NV = EW // 16
NS = NP // NT

BLK = 1280


def _proj_body(x_ref, w_ref, a2_ref, xp_ref, sd_ref):
    xp = lax.dot_general(x_ref[...], w_ref[...], (((1,), (1,)), ((), ())),
                         preferred_element_type=jnp.float32)
    xp_ref[...] = xp
    sd_ref[...] = lax.dot_general(xp, a2_ref[...], (((1,), (0,)), ((), ())),
                                  preferred_element_type=jnp.float32)


def _project(x_pad, w, a2):
    return pl.pallas_call(
        _proj_body,
        grid=(NP // BLK,),
        in_specs=[
            pl.BlockSpec((BLK, C), lambda i: (i, 0)),
            pl.BlockSpec((C, C), lambda i: (0, 0)),
            pl.BlockSpec((C, 2), lambda i: (0, 0)),
        ],
        out_specs=[
            pl.BlockSpec((BLK, C), lambda i: (i, 0)),
            pl.BlockSpec((BLK, 2), lambda i: (i, 0)),
        ],
        out_shape=[
            jax.ShapeDtypeStruct((NP, C), jnp.float32),
            jax.ShapeDtypeStruct((NP, 2), jnp.float32),
        ],
    )(x_pad, w, a2)


def _edge_body(s_hbm, d_hbm, row_hbm, col_hbm, s_out_hbm,
               s_v, d_v, row_v, col_v, ea_v, acc_v, tmp_a, tmp_b,
               part_sh, red_sh):
    tid = lax.axis_index("s")
    ebase = tid * EW
    nbase = tid * NS

    pltpu.sync_copy(s_hbm, s_v)
    pltpu.sync_copy(d_hbm, d_v)
    pltpu.sync_copy(row_hbm.at[pl.ds(ebase, EW)], row_v)
    pltpu.sync_copy(col_hbm.at[pl.ds(ebase, EW)], col_v)

    zeros16 = jnp.zeros((16,), jnp.float32)

    def zero_body(i, _):
        acc_v[pl.ds(i * 16, 16)] = zeros16
        return 0

    lax.fori_loop(0, NP // 16, zero_body, 0)

    def p1_body(i, _):
        b = i * 16
        r = row_v[pl.ds(b, 16)]
        c = col_v[pl.ds(b, 16)]
        a = plsc.load_gather(s_v, [r]) + plsc.load_gather(d_v, [c])
        a = jnp.where(a > 0, a, 0.2 * a)
        ea = jnp.exp(a)
        ea_v[pl.ds(b, 16)] = ea
        plsc.addupdate_scatter(acc_v, [r], ea - 1.0)
        return 0

    lax.fori_loop(0, NV, p1_body, 0)

    pltpu.sync_copy(acc_v, part_sh.at[tid])
    plsc.subcore_barrier()

    pltpu.sync_copy(part_sh.at[0, pl.ds(nbase, NS)], tmp_a)

    def red_body(t, _):
        pltpu.sync_copy(part_sh.at[t, pl.ds(nbase, NS)], tmp_b)

        def add_body(j, _):
            b = j * 16
            tmp_a[pl.ds(b, 16)] += tmp_b[pl.ds(b, 16)]
            return 0

        lax.fori_loop(0, NS // 16, add_body, 0)
        return 0

    lax.fori_loop(1, NT, red_body, 0)

    def adde_body(j, _):
        b = j * 16
        tmp_a[pl.ds(b, 16)] += jnp.float32(E)
        return 0

    lax.fori_loop(0, NS // 16, adde_body, 0)
    pltpu.sync_copy(tmp_a, red_sh.at[pl.ds(nbase, NS)])
    plsc.subcore_barrier()

    pltpu.sync_copy(red_sh, acc_v)

    def zero2_body(i, _):
        s_v[pl.ds(i * 16, 16)] = zeros16
        return 0

    lax.fori_loop(0, NP // 16, zero2_body, 0)

    def p2_body(i, _):
        b = i * 16
        r = row_v[pl.ds(b, 16)]
        c = col_v[pl.ds(b, 16)]
        an = ea_v[pl.ds(b, 16)] / plsc.load_gather(acc_v, [r])
        plsc.addupdate_scatter(s_v, [c], an)
        return 0

    lax.fori_loop(0, NV, p2_body, 0)

    pltpu.sync_copy(s_v, part_sh.at[tid])
    plsc.subcore_barrier()

    pltpu.sync_copy(part_sh.at[0, pl.ds(nbase, NS)], tmp_a)
    lax.fori_loop(1, NT, red_body, 0)
    pltpu.sync_copy(tmp_a, s_out_hbm.at[pl.ds(nbase, NS)])


def _edge_kernel(s, d, row, col):
    mesh = plsc.VectorSubcoreMesh(core_axis_name="c", subcore_axis_name="s",
                                  num_cores=1)
    return pl.kernel(
        _edge_body,
        mesh=mesh,
        out_type=jax.ShapeDtypeStruct((NP,), jnp.float32),
        scratch_types=[
            pltpu.VMEM((NP,), jnp.float32),
            pltpu.VMEM((NP,), jnp.float32),
            pltpu.VMEM((EW,), jnp.int32),
            pltpu.VMEM((EW,), jnp.int32),
            pltpu.VMEM((EW,), jnp.float32),
            pltpu.VMEM((NP,), jnp.float32),
            pltpu.VMEM((NS,), jnp.float32),
            pltpu.VMEM((NS,), jnp.float32),
            pltpu.VMEM_SHARED((NT, NP), jnp.float32),
            pltpu.VMEM_SHARED((NP,), jnp.float32),
        ],
    )(s, d, row, col)


def _scale_body(xp_ref, s_ref, b_ref, out_ref):
    out_ref[...] = xp_ref[...] * s_ref[...] + b_ref[...]


def _scale(xp, s_col, bias_row):
    return pl.pallas_call(
        _scale_body,
        grid=(NP // BLK,),
        in_specs=[
            pl.BlockSpec((BLK, C), lambda i: (i, 0)),
            pl.BlockSpec((BLK, 1), lambda i: (i, 0)),
            pl.BlockSpec((1, C), lambda i: (0, 0)),
        ],
        out_specs=pl.BlockSpec((BLK, C), lambda i: (i, 0)),
        out_shape=jax.ShapeDtypeStruct((NP, C), jnp.float32),
    )(xp, s_col, bias_row)


def kernel(x, W, att, bias, edge_index):
    x_pad = jnp.pad(x, ((0, NP - N), (0, 0)))
    a2 = jnp.stack([att[0, 0, :C], att[0, 0, C:]], axis=1)
    xp, sd = _project(x_pad, W, a2)
    row = edge_index[0]
    col = edge_index[1]
    s_sum = _edge_kernel(sd[:, 0], sd[:, 1], row, col)
    out = _scale(xp, s_sum[:, None], bias[None, :])
    return out[:N]


# speedup vs baseline: 39.8060x; 39.8060x over previous
import functools

import jax
import jax.numpy as jnp
from jax import lax
from jax.experimental import pallas as pl
from jax.experimental.pallas import tpu as pltpu
from jax.experimental.pallas import tpu_sc as plsc

N = 10000
E = 320000
C = 128

NP = 10240
NT = 16
EW = E // NT
NV = EW // 16
NS = NP // NT

BLK = 1280


def _proj_body(x_ref, w_ref, a2_ref, xp_ref, sd_ref):
    xp = lax.dot_general(x_ref[...], w_ref[...], (((1,), (1,)), ((), ())),
                         preferred_element_type=jnp.float32)
    xp_ref[...] = xp
    sd_ref[...] = lax.dot_general(xp, a2_ref[...], (((1,), (0,)), ((), ())),
                                  preferred_element_type=jnp.float32)


def _project(x_pad, w, a2):
    return pl.pallas_call(
        _proj_body,
        grid=(NP // BLK,),
        in_specs=[
            pl.BlockSpec((BLK, C), lambda i: (i, 0)),
            pl.BlockSpec((C, C), lambda i: (0, 0)),
            pl.BlockSpec((C, 2), lambda i: (0, 0)),
        ],
        out_specs=[
            pl.BlockSpec((BLK, C), lambda i: (i, 0)),
            pl.BlockSpec((BLK, 2), lambda i: (i, 0)),
        ],
        out_shape=[
            jax.ShapeDtypeStruct((NP, C), jnp.float32),
            jax.ShapeDtypeStruct((NP, 2), jnp.float32),
        ],
    )(x_pad, w, a2)


def _edge_body(s_hbm, d_hbm, row_hbm, col_hbm, s_out_hbm,
               s_v, d_v, row_v, col_v, ea_v, acc_v, tmp_a, tmp_b,
               part_sh, red_sh):
    tid = lax.axis_index("s")
    ebase = tid * EW
    nbase = tid * NS

    pltpu.sync_copy(s_hbm, s_v)
    pltpu.sync_copy(d_hbm, d_v)
    pltpu.sync_copy(row_hbm.at[pl.ds(ebase, EW)], row_v)
    pltpu.sync_copy(col_hbm.at[pl.ds(ebase, EW)], col_v)

    zeros16 = jnp.zeros((16,), jnp.float32)

    def zero_body(i, _):
        acc_v[pl.ds(i * 16, 16)] = zeros16
        return 0

    lax.fori_loop(0, NP // 16, zero_body, 0)

    def p1_body(i, _):
        b = i * 16
        r = row_v[pl.ds(b, 16)]
        c = col_v[pl.ds(b, 16)]
        a = plsc.load_gather(s_v, [r]) + plsc.load_gather(d_v, [c])
        a = jnp.where(a > 0, a, 0.2 * a)
        ea = jnp.exp(a)
        ea_v[pl.ds(b, 16)] = ea
        plsc.addupdate_scatter(acc_v, [r], ea - 1.0)
        return 0

    lax.fori_loop(0, NV, p1_body, 0)

    pltpu.sync_copy(acc_v, part_sh.at[tid])
    plsc.subcore_barrier()

    pltpu.sync_copy(part_sh.at[0, pl.ds(nbase, NS)], tmp_a)

    def red_body(t, _):
        pltpu.sync_copy(part_sh.at[t, pl.ds(nbase, NS)], tmp_b)

        def add_body(j, _):
            b = j * 16
            tmp_a[pl.ds(b, 16)] += tmp_b[pl.ds(b, 16)]
            return 0

        lax.fori_loop(0, NS // 16, add_body, 0)
        return 0

    lax.fori_loop(1, NT, red_body, 0)

    def adde_body(j, _):
        b = j * 16
        tmp_a[pl.ds(b, 16)] += jnp.float32(E)
        return 0

    lax.fori_loop(0, NS // 16, adde_body, 0)
    pltpu.sync_copy(tmp_a, red_sh.at[pl.ds(nbase, NS)])
    plsc.subcore_barrier()

    pltpu.sync_copy(red_sh, acc_v)

    def zero2_body(i, _):
        s_v[pl.ds(i * 16, 16)] = zeros16
        return 0

    lax.fori_loop(0, NP // 16, zero2_body, 0)

    def p2_body(i, _):
        b = i * 16
        r = row_v[pl.ds(b, 16)]
        c = col_v[pl.ds(b, 16)]
        an = ea_v[pl.ds(b, 16)] / plsc.load_gather(acc_v, [r])
        plsc.addupdate_scatter(s_v, [c], an)
        return 0

    lax.fori_loop(0, NV, p2_body, 0)

    pltpu.sync_copy(s_v, part_sh.at[tid])
    plsc.subcore_barrier()

    pltpu.sync_copy(part_sh.at[0, pl.ds(nbase, NS)], tmp_a)
    lax.fori_loop(1, NT, red_body, 0)
    pltpu.sync_copy(tmp_a, s_out_hbm.at[pl.ds(nbase, NS)])


def _edge_kernel(s, d, row, col):
    mesh = plsc.VectorSubcoreMesh(core_axis_name="c", subcore_axis_name="s",
                                  num_cores=1)
    return pl.kernel(
        _edge_body,
        mesh=mesh,
        compiler_params=pltpu.CompilerParams(needs_layout_passes=False),
        out_type=jax.ShapeDtypeStruct((NP,), jnp.float32),
        scratch_types=[
            pltpu.VMEM((NP,), jnp.float32),
            pltpu.VMEM((NP,), jnp.float32),
            pltpu.VMEM((EW,), jnp.int32),
            pltpu.VMEM((EW,), jnp.int32),
            pltpu.VMEM((EW,), jnp.float32),
            pltpu.VMEM((NP,), jnp.float32),
            pltpu.VMEM((NS,), jnp.float32),
            pltpu.VMEM((NS,), jnp.float32),
            pltpu.VMEM_SHARED((NT, NP), jnp.float32),
            pltpu.VMEM_SHARED((NP,), jnp.float32),
        ],
    )(s, d, row, col)


def _scale_body(xp_ref, s_ref, b_ref, out_ref):
    out_ref[...] = xp_ref[...] * s_ref[...] + b_ref[...]


def _scale(xp, s_col, bias_row):
    return pl.pallas_call(
        _scale_body,
        grid=(NP // BLK,),
        in_specs=[
            pl.BlockSpec((BLK, C), lambda i: (i, 0)),
            pl.BlockSpec((BLK, 1), lambda i: (i, 0)),
            pl.BlockSpec((1, C), lambda i: (0, 0)),
        ],
        out_specs=pl.BlockSpec((BLK, C), lambda i: (i, 0)),
        out_shape=jax.ShapeDtypeStruct((NP, C), jnp.float32),
    )(xp, s_col, bias_row)


def kernel(x, W, att, bias, edge_index):
    x_pad = jnp.pad(x, ((0, NP - N), (0, 0)))
    a2 = jnp.stack([att[0, 0, :C], att[0, 0, C:]], axis=1)
    xp, sd = _project(x_pad, W, a2)
    row = edge_index[0]
    col = edge_index[1]
    s_sum = _edge_kernel(sd[:, 0], sd[:, 1], row, col)
    out = _scale(xp, s_sum[:, None], bias[None, :])
    return out[:N]


# speedup vs baseline: 43.4147x; 1.0907x over previous
import functools

import jax
import jax.numpy as jnp
from jax import lax
from jax.experimental import pallas as pl
from jax.experimental.pallas import tpu as pltpu
from jax.experimental.pallas import tpu_sc as plsc

N = 10000
E = 320000
C = 128

NP = 10240
NT = 16
EW = E // NT
NV = EW // 16
NS = NP // NT

BLK = 1000


def _proj_body(x_ref, w_ref, a2_ref, xp_ref, s_ref, d_ref):
    xp = lax.dot_general(x_ref[...], w_ref[...], (((1,), (1,)), ((), ())),
                         preferred_element_type=jnp.float32)
    xp_ref[...] = xp
    sd = lax.dot_general(xp, a2_ref[...], (((1,), (0,)), ((), ())),
                         preferred_element_type=jnp.float32)
    s_ref[...] = sd[:, 0:1]
    d_ref[...] = sd[:, 1:2]


def _project(x, w, a2):
    return pl.pallas_call(
        _proj_body,
        grid=(N // BLK,),
        in_specs=[
            pl.BlockSpec((BLK, C), lambda i: (i, 0)),
            pl.BlockSpec((C, C), lambda i: (0, 0)),
            pl.BlockSpec((C, 2), lambda i: (0, 0)),
        ],
        out_specs=[
            pl.BlockSpec((BLK, C), lambda i: (i, 0)),
            pl.BlockSpec((BLK, 1), lambda i: (i, 0)),
            pl.BlockSpec((BLK, 1), lambda i: (i, 0)),
        ],
        out_shape=[
            jax.ShapeDtypeStruct((N, C), jnp.float32),
            jax.ShapeDtypeStruct((N, 1), jnp.float32),
            jax.ShapeDtypeStruct((N, 1), jnp.float32),
        ],
    )(x, w, a2)


def _edge_body(s_hbm, d_hbm, row_hbm, col_hbm, s_out_hbm,
               s_v, d_v, row_v, col_v, ea_v, acc_v, red_v, part_sh, red_sh,
               sem):
    tid = lax.axis_index("s")
    ebase = tid * EW
    nbase = tid * NS

    cp = [
        pltpu.make_async_copy(s_hbm, s_v.at[pl.ds(0, N)], sem),
        pltpu.make_async_copy(d_hbm, d_v.at[pl.ds(0, N)], sem),
        pltpu.make_async_copy(row_hbm.at[pl.ds(ebase, EW)], row_v, sem),
        pltpu.make_async_copy(col_hbm.at[pl.ds(ebase, EW)], col_v, sem),
    ]
    for c in cp:
        c.start()

    zeros16 = jnp.zeros((16,), jnp.float32)

    def zero_body(i, _):
        acc_v[pl.ds(i * 16, 16)] = zeros16
        return 0

    lax.fori_loop(0, NP // 16, zero_body, 0, unroll=8)
    for c in cp:
        c.wait()

    def p1_body(i, _):
        b = i * 16
        r = row_v[pl.ds(b, 16)]
        c = col_v[pl.ds(b, 16)]
        a = plsc.load_gather(s_v, [r]) + plsc.load_gather(d_v, [c])
        a = jnp.where(a > 0, a, 0.2 * a)
        ea = jnp.exp(a)
        ea_v[pl.ds(b, 16)] = ea
        plsc.addupdate_scatter(acc_v, [r], ea - 1.0)
        return 0

    lax.fori_loop(0, NV, p1_body, 0, unroll=8)

    pltpu.sync_copy(acc_v, part_sh.at[tid])
    plsc.subcore_barrier()

    pltpu.sync_copy(part_sh.at[:, pl.ds(nbase, NS)], red_v)

    def red_body(j, _):
        b = j * 16
        v = red_v[0, pl.ds(b, 16)]
        for t in range(1, NT):
            v = v + red_v[t, pl.ds(b, 16)]
        acc_v[pl.ds(b, 16)] = v + jnp.float32(E)
        return 0

    lax.fori_loop(0, NS // 16, red_body, 0)
    pltpu.sync_copy(acc_v.at[pl.ds(0, NS)], red_sh.at[pl.ds(nbase, NS)])
    plsc.subcore_barrier()

    pltpu.sync_copy(red_sh, acc_v)

    def zero2_body(i, _):
        s_v[pl.ds(i * 16, 16)] = zeros16
        return 0

    lax.fori_loop(0, NP // 16, zero2_body, 0, unroll=8)

    def p2_body(i, _):
        b = i * 16
        r = row_v[pl.ds(b, 16)]
        c = col_v[pl.ds(b, 16)]
        an = ea_v[pl.ds(b, 16)] / plsc.load_gather(acc_v, [r])
        plsc.addupdate_scatter(s_v, [c], an)
        return 0

    lax.fori_loop(0, NV, p2_body, 0, unroll=8)

    pltpu.sync_copy(s_v, part_sh.at[tid])
    plsc.subcore_barrier()

    pltpu.sync_copy(part_sh.at[:, pl.ds(nbase, NS)], red_v)

    def red2_body(j, _):
        b = j * 16
        v = red_v[0, pl.ds(b, 16)]
        for t in range(1, NT):
            v = v + red_v[t, pl.ds(b, 16)]
        acc_v[pl.ds(b, 16)] = v
        return 0

    lax.fori_loop(0, NS // 16, red2_body, 0)
    pltpu.sync_copy(acc_v.at[pl.ds(0, NS)], s_out_hbm.at[pl.ds(nbase, NS)])


def _edge_kernel(s, d, row, col):
    mesh = plsc.VectorSubcoreMesh(core_axis_name="c", subcore_axis_name="s",
                                  num_cores=1)
    return pl.kernel(
        _edge_body,
        mesh=mesh,
        compiler_params=pltpu.CompilerParams(needs_layout_passes=False),
        out_type=jax.ShapeDtypeStruct((NP,), jnp.float32),
        scratch_types=[
            pltpu.VMEM((NP,), jnp.float32),
            pltpu.VMEM((NP,), jnp.float32),
            pltpu.VMEM((EW,), jnp.int32),
            pltpu.VMEM((EW,), jnp.int32),
            pltpu.VMEM((EW,), jnp.float32),
            pltpu.VMEM((NP,), jnp.float32),
            pltpu.VMEM((NT, NS), jnp.float32),
            pltpu.VMEM_SHARED((NT, NP), jnp.float32),
            pltpu.VMEM_SHARED((NP,), jnp.float32),
            pltpu.SemaphoreType.DMA,
        ],
    )(s, d, row, col)


def _scale_body(xp_ref, s_ref, b_ref, out_ref):
    out_ref[...] = xp_ref[...] * s_ref[...] + b_ref[...]


def _scale(xp, s_col, bias_row):
    return pl.pallas_call(
        _scale_body,
        grid=(N // BLK,),
        in_specs=[
            pl.BlockSpec((BLK, C), lambda i: (i, 0)),
            pl.BlockSpec((BLK, 1), lambda i: (i, 0)),
            pl.BlockSpec((1, C), lambda i: (0, 0)),
        ],
        out_specs=pl.BlockSpec((BLK, C), lambda i: (i, 0)),
        out_shape=jax.ShapeDtypeStruct((N, C), jnp.float32),
    )(xp, s_col, bias_row)


def kernel(x, W, att, bias, edge_index):
    a2 = jnp.stack([att[0, 0, :C], att[0, 0, C:]], axis=1)
    xp, s1, d1 = _project(x, W, a2)
    row = edge_index[0]
    col = edge_index[1]
    s_sum = _edge_kernel(s1.reshape(N), d1.reshape(N), row, col)
    out = _scale(xp, s_sum[:N, None], bias[None, :])
    return out


# speedup vs baseline: 78.7574x; 1.8141x over previous
import functools

import jax
import jax.numpy as jnp
from jax import lax
from jax.experimental import pallas as pl
from jax.experimental.pallas import tpu as pltpu
from jax.experimental.pallas import tpu_sc as plsc

N = 10000
E = 320000
C = 128

NP = 10240
NT = 16
NC = 2
NW = NT * NC
EW = E // NW
NV = EW // 16
NS = NP // NT

BLK = 1024
NR = NP // 128


def _sd_body(x_ref, w_ref, a2_ref, s_ref, d_ref):
    wa = lax.dot_general(w_ref[...], a2_ref[...], (((0,), (0,)), ((), ())),
                         preferred_element_type=jnp.float32)
    sd = lax.dot_general(x_ref[...], wa, (((1,), (0,)), ((), ())),
                         preferred_element_type=jnp.float32)
    s_ref[...] = sd[:, 0].reshape(BLK // 128, 128)
    d_ref[...] = sd[:, 1].reshape(BLK // 128, 128)


def _project_sd(x, w, a2):
    return pl.pallas_call(
        _sd_body,
        grid=(NP // BLK,),
        in_specs=[
            pl.BlockSpec((BLK, C), lambda i: (i, 0)),
            pl.BlockSpec((C, C), lambda i: (0, 0)),
            pl.BlockSpec((C, 2), lambda i: (0, 0)),
        ],
        out_specs=[
            pl.BlockSpec((BLK // 128, 128), lambda i: (i, 0)),
            pl.BlockSpec((BLK // 128, 128), lambda i: (i, 0)),
        ],
        out_shape=[
            jax.ShapeDtypeStruct((NR, 128), jnp.float32),
            jax.ShapeDtypeStruct((NR, 128), jnp.float32),
        ],
    )(x, w, a2)


def _xp_body(x_ref, w_ref, xp_ref):
    xp_ref[...] = lax.dot_general(x_ref[...], w_ref[...],
                                  (((1,), (1,)), ((), ())),
                                  preferred_element_type=jnp.float32)


def _matmul_xp(x, w):
    return pl.pallas_call(
        _xp_body,
        grid=(NP // BLK,),
        in_specs=[
            pl.BlockSpec((BLK, C), lambda i: (i, 0)),
            pl.BlockSpec((C, C), lambda i: (0, 0)),
        ],
        out_specs=pl.BlockSpec((BLK, C), lambda i: (i, 0)),
        out_shape=jax.ShapeDtypeStruct((N, C), jnp.float32),
    )(x, w)


def _p1_body(s_hbm, d_hbm, ei_hbm, p_out_hbm, ea_out_hbm,
             s_v, d_v, row_v, col_v, ea_v, acc_v, red_v, part_sh, sem):
    cid = lax.axis_index("c")
    sid = lax.axis_index("s")
    wid = sid * NC + cid
    ebase = wid * EW
    nbase = sid * NS

    cp = [
        pltpu.make_async_copy(s_hbm, s_v, sem),
        pltpu.make_async_copy(d_hbm, d_v, sem),
        pltpu.make_async_copy(ei_hbm.at[pl.ds(ebase, EW)], row_v, sem),
        pltpu.make_async_copy(ei_hbm.at[pl.ds(E + ebase, EW)], col_v, sem),
    ]
    for c in cp:
        c.start()

    zeros16 = jnp.zeros((16,), jnp.float32)

    @plsc.parallel_loop(0, NP // 16, 1, unroll=8)
    def zero_body(i):
        acc_v[pl.ds(i * 16, 16)] = zeros16

    for c in cp:
        c.wait()

    @plsc.parallel_loop(0, NV, 1, unroll=8)
    def p1_body(i):
        b = i * 16
        r = row_v[pl.ds(b, 16)]
        c = col_v[pl.ds(b, 16)]
        a = plsc.load_gather(s_v, [r]) + plsc.load_gather(d_v, [c])
        a = jnp.where(a > 0, a, 0.2 * a)
        ea = jnp.exp(a)
        ea_v[pl.ds(b, 16)] = ea
        plsc.addupdate_scatter(acc_v, [r], ea - 1.0)

    ea_cp = pltpu.make_async_copy(ea_v, ea_out_hbm.at[pl.ds(ebase, EW)], sem)
    ea_cp.start()

    pltpu.sync_copy(acc_v, part_sh.at[sid])
    plsc.subcore_barrier()
    pltpu.sync_copy(part_sh.at[:, pl.ds(nbase, NS)], red_v)

    @plsc.parallel_loop(0, NS // 16, 1, unroll=4)
    def red_body(j):
        b = j * 16
        v = red_v[0, pl.ds(b, 16)]
        for t in range(1, NT):
            v = v + red_v[t, pl.ds(b, 16)]
        acc_v[pl.ds(b, 16)] = v

    pltpu.sync_copy(acc_v.at[pl.ds(0, NS)],
                    p_out_hbm.at[pl.ds(cid * NP + nbase, NS)])
    ea_cp.wait()


def _edge_p1(s, d, ei):
    mesh = plsc.VectorSubcoreMesh(core_axis_name="c", subcore_axis_name="s",
                                  num_cores=NC)
    return pl.kernel(
        _p1_body,
        mesh=mesh,
        compiler_params=pltpu.CompilerParams(needs_layout_passes=False),
        out_type=[
            jax.ShapeDtypeStruct((NC * NP,), jnp.float32),
            jax.ShapeDtypeStruct((E,), jnp.float32),
        ],
        scratch_types=[
            pltpu.VMEM((NP,), jnp.float32),
            pltpu.VMEM((NP,), jnp.float32),
            pltpu.VMEM((EW,), jnp.int32),
            pltpu.VMEM((EW,), jnp.int32),
            pltpu.VMEM((EW,), jnp.float32),
            pltpu.VMEM((NP,), jnp.float32),
            pltpu.VMEM((NT, NS), jnp.float32),
            pltpu.VMEM_SHARED((NT, NP), jnp.float32),
            pltpu.SemaphoreType.DMA,
        ],
    )(s, d, ei)


def _p2_body(ei_hbm, ea_hbm, p_hbm, s_out_hbm,
             acc_v, tmp_v, row_v, col_v, ea_v, red_v, part_sh, sem):
    cid = lax.axis_index("c")
    sid = lax.axis_index("s")
    wid = sid * NC + cid
    ebase = wid * EW
    nbase = sid * NS

    cp = [
        pltpu.make_async_copy(p_hbm.at[pl.ds(0, NP)], acc_v, sem),
        pltpu.make_async_copy(p_hbm.at[pl.ds(NP, NP)], tmp_v, sem),
        pltpu.make_async_copy(ei_hbm.at[pl.ds(ebase, EW)], row_v, sem),
        pltpu.make_async_copy(ei_hbm.at[pl.ds(E + ebase, EW)], col_v, sem),
        pltpu.make_async_copy(ea_hbm.at[pl.ds(ebase, EW)], ea_v, sem),
    ]
    for c in cp:
        c.start()
    for c in cp:
        c.wait()

    fE = jnp.float32(E)

    @plsc.parallel_loop(0, NP // 16, 1, unroll=8)
    def denom_body(i):
        b = i * 16
        acc_v[pl.ds(b, 16)] = acc_v[pl.ds(b, 16)] + tmp_v[pl.ds(b, 16)] + fE

    zeros16 = jnp.zeros((16,), jnp.float32)

    @plsc.parallel_loop(0, NP // 16, 1, unroll=8)
    def zero_body(i):
        tmp_v[pl.ds(i * 16, 16)] = zeros16

    @plsc.parallel_loop(0, NV, 1, unroll=8)
    def p2_body(i):
        b = i * 16
        r = row_v[pl.ds(b, 16)]
        c = col_v[pl.ds(b, 16)]
        an = ea_v[pl.ds(b, 16)] / plsc.load_gather(acc_v, [r])
        plsc.addupdate_scatter(tmp_v, [c], an)

    pltpu.sync_copy(tmp_v, part_sh.at[sid])
    plsc.subcore_barrier()
    pltpu.sync_copy(part_sh.at[:, pl.ds(nbase, NS)], red_v)

    @plsc.parallel_loop(0, NS // 16, 1, unroll=4)
    def red_body(j):
        b = j * 16
        v = red_v[0, pl.ds(b, 16)]
        for t in range(1, NT):
            v = v + red_v[t, pl.ds(b, 16)]
        acc_v[pl.ds(b, 16)] = v

    pltpu.sync_copy(acc_v.at[pl.ds(0, NS)],
                    s_out_hbm.at[pl.ds(cid * NP + nbase, NS)])


def _edge_p2(ei, ea, p):
    mesh = plsc.VectorSubcoreMesh(core_axis_name="c", subcore_axis_name="s",
                                  num_cores=NC)
    return pl.kernel(
        _p2_body,
        mesh=mesh,
        compiler_params=pltpu.CompilerParams(needs_layout_passes=False),
        out_type=jax.ShapeDtypeStruct((NC * NP,), jnp.float32),
        scratch_types=[
            pltpu.VMEM((NP,), jnp.float32),
            pltpu.VMEM((NP,), jnp.float32),
            pltpu.VMEM((EW,), jnp.int32),
            pltpu.VMEM((EW,), jnp.int32),
            pltpu.VMEM((EW,), jnp.float32),
            pltpu.VMEM((NT, NS), jnp.float32),
            pltpu.VMEM_SHARED((NT, NP), jnp.float32),
            pltpu.SemaphoreType.DMA,
        ],
    )(ei, ea, p)


def _scale_body(xp_ref, s_ref, b_ref, out_ref):
    ir = lax.broadcasted_iota(jnp.int32, (128, 128), 0)
    ic = lax.broadcasted_iota(jnp.int32, (128, 128), 1)
    eye = (ir == ic)
    s_blk = s_ref[0] + s_ref[1]
    for g in range(BLK // 128):
        srow = s_blk[g:g + 1, :]
        dg = jnp.where(eye, jnp.broadcast_to(srow, (128, 128)), 0.0)
        xp_g = xp_ref[pl.ds(g * 128, 128), :]
        out_ref[pl.ds(g * 128, 128), :] = (
            lax.dot_general(dg, xp_g, (((1,), (0,)), ((), ())),
                            preferred_element_type=jnp.float32)
            + b_ref[...]
        )


def _scale(xp, s2, bias_row):
    return pl.pallas_call(
        _scale_body,
        grid=(NP // BLK,),
        in_specs=[
            pl.BlockSpec((BLK, C), lambda i: (i, 0)),
            pl.BlockSpec((NC, BLK // 128, 128), lambda i: (0, i, 0)),
            pl.BlockSpec((1, C), lambda i: (0, 0)),
        ],
        out_specs=pl.BlockSpec((BLK, C), lambda i: (i, 0)),
        out_shape=jax.ShapeDtypeStruct((N, C), jnp.float32),
    )(xp, s2, bias_row)


def kernel(x, W, att, bias, edge_index):
    a2 = jnp.stack([att[0, 0, :C], att[0, 0, C:]], axis=1)
    s80, d80 = _project_sd(x, W, a2)
    ei = edge_index.reshape(2 * E)
    p, ea = _edge_p1(s80.reshape(NP), d80.reshape(NP), ei)
    xp = _matmul_xp(x, W)
    s2 = _edge_p2(ei, ea, p)
    out = _scale(xp, s2.reshape(NC, NR, 128), bias[None, :])
    return out


# speedup vs baseline: 90.5741x; 1.1500x over previous
import functools

import jax
import jax.numpy as jnp
from jax import lax
from jax.experimental import pallas as pl
from jax.experimental.pallas import tpu as pltpu
from jax.experimental.pallas import tpu_sc as plsc

N = 10000
E = 320000
C = 128

NP = 10240
NT = 16
NC = 2
EW = E // NT
NV = EW // 16
NV2 = NV // NC
NS = NP // NT

BLK = 1024
NR = NP // 128


def _sd_body(x_ref, w_ref, a2_ref, sd_ref):
    wa = lax.dot_general(a2_ref[...], w_ref[...], (((0,), (0,)), ((), ())),
                         preferred_element_type=jnp.float32)
    sd = lax.dot_general(wa, x_ref[...], (((1,), (1,)), ((), ())),
                         preferred_element_type=jnp.float32)
    sd_ref[...] = sd.reshape(2, BLK // 128, 128)


def _project_sd(x, w, a2):
    return pl.pallas_call(
        _sd_body,
        grid=(NP // BLK,),
        in_specs=[
            pl.BlockSpec((BLK, C), lambda i: (i, 0)),
            pl.BlockSpec((C, C), lambda i: (0, 0)),
            pl.BlockSpec((C, 2), lambda i: (0, 0)),
        ],
        out_specs=pl.BlockSpec((2, BLK // 128, 128), lambda i: (0, i, 0)),
        out_shape=jax.ShapeDtypeStruct((2, NR, 128), jnp.float32),
    )(x, w, a2)


def _xp_body(x_ref, w_ref, xp_ref):
    xp_ref[...] = lax.dot_general(x_ref[...], w_ref[...],
                                  (((1,), (1,)), ((), ())),
                                  preferred_element_type=jnp.float32)


def _matmul_xp(x, w):
    return pl.pallas_call(
        _xp_body,
        grid=(NP // BLK,),
        in_specs=[
            pl.BlockSpec((BLK, C), lambda i: (i, 0)),
            pl.BlockSpec((C, C), lambda i: (0, 0)),
        ],
        out_specs=pl.BlockSpec((BLK, C), lambda i: (i, 0)),
        out_shape=jax.ShapeDtypeStruct((N, C), jnp.float32),
    )(x, w)


def _edge_body(sd_hbm, ei_hbm, s_out_hbm,
               s_v, d_v, row_v, col_v, ea_v, acc_v, red_v, part_sh, red_sh,
               sem):
    cid = lax.axis_index("c")
    sid = lax.axis_index("s")
    ebase = sid * EW
    nbase = sid * NS

    cp = [
        pltpu.make_async_copy(sd_hbm.at[pl.ds(0, NP)], s_v, sem),
        pltpu.make_async_copy(sd_hbm.at[pl.ds(NP, NP)], d_v, sem),
        pltpu.make_async_copy(ei_hbm.at[pl.ds(ebase, EW)], row_v, sem),
        pltpu.make_async_copy(ei_hbm.at[pl.ds(E + ebase, EW)], col_v, sem),
    ]
    for c in cp:
        c.start()

    zeros16 = jnp.zeros((16,), jnp.float32)

    @plsc.parallel_loop(0, NP // 16, 1, unroll=8)
    def zero_body(i):
        acc_v[pl.ds(i * 16, 16)] = zeros16

    for c in cp:
        c.wait()

    @plsc.parallel_loop(0, NV, 1, unroll=8)
    def p1_body(i):
        b = i * 16
        r = row_v[pl.ds(b, 16)]
        c = col_v[pl.ds(b, 16)]
        a = plsc.load_gather(s_v, [r]) + plsc.load_gather(d_v, [c])
        a = jnp.where(a > 0, a, 0.2 * a)
        ea = jnp.exp(a)
        ea_v[pl.ds(b, 16)] = ea
        plsc.addupdate_scatter(acc_v, [r], ea - 1.0)

    pltpu.sync_copy(acc_v, part_sh.at[sid])
    plsc.subcore_barrier()
    pltpu.sync_copy(part_sh.at[:, pl.ds(nbase, NS)], red_v)

    fE = jnp.float32(E)

    @plsc.parallel_loop(0, NS // 16, 1, unroll=4)
    def red_body(j):
        b = j * 16
        v = red_v[0, pl.ds(b, 16)]
        for t in range(1, NT):
            v = v + red_v[t, pl.ds(b, 16)]
        acc_v[pl.ds(b, 16)] = v + fE

    pltpu.sync_copy(acc_v.at[pl.ds(0, NS)], red_sh.at[pl.ds(nbase, NS)])
    plsc.subcore_barrier()

    pltpu.sync_copy(red_sh, acc_v)

    @plsc.parallel_loop(0, NP // 16, 1, unroll=8)
    def zero2_body(i):
        s_v[pl.ds(i * 16, 16)] = zeros16

    half = cid * NV2 * 16

    @plsc.parallel_loop(0, NV2, 1, unroll=8)
    def p2_body(i):
        b = half + i * 16
        r = row_v[pl.ds(b, 16)]
        c = col_v[pl.ds(b, 16)]
        an = ea_v[pl.ds(b, 16)] / plsc.load_gather(acc_v, [r])
        plsc.addupdate_scatter(s_v, [c], an)

    pltpu.sync_copy(s_v, part_sh.at[sid])
    plsc.subcore_barrier()
    pltpu.sync_copy(part_sh.at[:, pl.ds(nbase, NS)], red_v)

    @plsc.parallel_loop(0, NS // 16, 1, unroll=4)
    def red2_body(j):
        b = j * 16
        v = red_v[0, pl.ds(b, 16)]
        for t in range(1, NT):
            v = v + red_v[t, pl.ds(b, 16)]
        acc_v[pl.ds(b, 16)] = v

    pltpu.sync_copy(acc_v.at[pl.ds(0, NS)],
                    s_out_hbm.at[pl.ds(cid * NP + nbase, NS)])


def _edge_kernel(sd, ei):
    mesh = plsc.VectorSubcoreMesh(core_axis_name="c", subcore_axis_name="s",
                                  num_cores=NC)
    return pl.kernel(
        _edge_body,
        mesh=mesh,
        compiler_params=pltpu.CompilerParams(needs_layout_passes=False),
        out_type=jax.ShapeDtypeStruct((NC * NP,), jnp.float32),
        scratch_types=[
            pltpu.VMEM((NP,), jnp.float32),
            pltpu.VMEM((NP,), jnp.float32),
            pltpu.VMEM((EW,), jnp.int32),
            pltpu.VMEM((EW,), jnp.int32),
            pltpu.VMEM((EW,), jnp.float32),
            pltpu.VMEM((NP,), jnp.float32),
            pltpu.VMEM((NT, NS), jnp.float32),
            pltpu.VMEM_SHARED((NT, NP), jnp.float32),
            pltpu.VMEM_SHARED((NP,), jnp.float32),
            pltpu.SemaphoreType.DMA,
        ],
    )(sd, ei)


def _scale_body(xp_ref, s_ref, b_ref, out_ref):
    ir = lax.broadcasted_iota(jnp.int32, (128, 128), 0)
    ic = lax.broadcasted_iota(jnp.int32, (128, 128), 1)
    eye = (ir == ic)
    s_blk = s_ref[0] + s_ref[1]
    for g in range(BLK // 128):
        srow = s_blk[g:g + 1, :]
        dg = jnp.where(eye, jnp.broadcast_to(srow, (128, 128)), 0.0)
        xp_g = xp_ref[pl.ds(g * 128, 128), :]
        out_ref[pl.ds(g * 128, 128), :] = (
            lax.dot_general(dg, xp_g, (((1,), (0,)), ((), ())),
                            preferred_element_type=jnp.float32)
            + b_ref[...]
        )


def _scale(xp, s2, bias_row):
    return pl.pallas_call(
        _scale_body,
        grid=(NP // BLK,),
        in_specs=[
            pl.BlockSpec((BLK, C), lambda i: (i, 0)),
            pl.BlockSpec((NC, BLK // 128, 128), lambda i: (0, i, 0)),
            pl.BlockSpec((1, C), lambda i: (0, 0)),
        ],
        out_specs=pl.BlockSpec((BLK, C), lambda i: (i, 0)),
        out_shape=jax.ShapeDtypeStruct((N, C), jnp.float32),
    )(xp, s2, bias_row)


def kernel(x, W, att, bias, edge_index):
    a2 = jnp.stack([att[0, 0, :C], att[0, 0, C:]], axis=1)
    sd = _project_sd(x, W, a2)
    ei = edge_index.reshape(2 * E)
    s2 = _edge_kernel(sd.reshape(NC * NP), ei)
    xp = _matmul_xp(x, W)
    out = _scale(xp, s2.reshape(NC, NR, 128), bias[None, :])
    return out


# speedup vs baseline: 90.6355x; 1.0007x over previous
import functools

import jax
import jax.numpy as jnp
from jax import lax
from jax.experimental import pallas as pl
from jax.experimental.pallas import tpu as pltpu
from jax.experimental.pallas import tpu_sc as plsc

N = 10000
E = 320000
C = 128

NP = 10240
NT = 16
NC = 2
EW = E // NT
NV = EW // 16
NV2 = NV // NC
NS = NP // NT

BLK = 1024
NR = NP // 128


def _sd_body(x_ref, w_ref, a2_ref, sd_ref):
    wa = lax.dot_general(a2_ref[...], w_ref[...], (((0,), (0,)), ((), ())),
                         preferred_element_type=jnp.float32)
    sd = lax.dot_general(wa, x_ref[...], (((1,), (1,)), ((), ())),
                         preferred_element_type=jnp.float32)
    sd_ref[...] = sd.reshape(2, BLK // 128, 128)


def _project_sd(x, w, a2):
    return pl.pallas_call(
        _sd_body,
        grid=(NP // BLK,),
        in_specs=[
            pl.BlockSpec((BLK, C), lambda i: (i, 0)),
            pl.BlockSpec((C, C), lambda i: (0, 0)),
            pl.BlockSpec((C, 2), lambda i: (0, 0)),
        ],
        out_specs=pl.BlockSpec((2, BLK // 128, 128), lambda i: (0, i, 0)),
        out_shape=jax.ShapeDtypeStruct((2, NR, 128), jnp.float32),
    )(x, w, a2)


def _xp_body(x_ref, w_ref, xp_ref):
    xp_ref[...] = lax.dot_general(x_ref[...], w_ref[...],
                                  (((1,), (1,)), ((), ())),
                                  preferred_element_type=jnp.float32)


def _matmul_xp(x, w):
    return pl.pallas_call(
        _xp_body,
        grid=(NP // BLK,),
        in_specs=[
            pl.BlockSpec((BLK, C), lambda i: (i, 0)),
            pl.BlockSpec((C, C), lambda i: (0, 0)),
        ],
        out_specs=pl.BlockSpec((BLK, C), lambda i: (i, 0)),
        out_shape=jax.ShapeDtypeStruct((N, C), jnp.float32),
    )(x, w)


def _edge_body(sd_hbm, ei_hbm, s_out_hbm,
               s_v, d_v, row_v, col_v, ea_v, acc_v, red_v, part_sh, red_sh,
               sem):
    cid = lax.axis_index("c")
    sid = lax.axis_index("s")
    ebase = sid * EW
    nbase = sid * NS

    cp = [
        pltpu.make_async_copy(sd_hbm.at[pl.ds(0, NP)], s_v, sem),
        pltpu.make_async_copy(sd_hbm.at[pl.ds(NP, NP)], d_v, sem),
        pltpu.make_async_copy(ei_hbm.at[pl.ds(ebase, EW)], row_v, sem),
        pltpu.make_async_copy(ei_hbm.at[pl.ds(E + ebase, EW)], col_v, sem),
    ]
    for c in cp:
        c.start()

    zeros16 = jnp.zeros((16,), jnp.float32)

    @plsc.parallel_loop(0, NP // 16, 1, unroll=8)
    def zero_body(i):
        acc_v[pl.ds(i * 16, 16)] = zeros16

    for c in cp:
        c.wait()

    @plsc.parallel_loop(0, NV, 1, unroll=8)
    def p1_body(i):
        b = i * 16
        r = row_v[pl.ds(b, 16)]
        c = col_v[pl.ds(b, 16)]
        a = plsc.load_gather(s_v, [r]) + plsc.load_gather(d_v, [c])
        a = jnp.where(a > 0, a, 0.2 * a)
        ea = jnp.exp(a)
        ea_v[pl.ds(b, 16)] = ea
        plsc.addupdate_scatter(acc_v, [r], ea - 1.0)

    pltpu.sync_copy(acc_v, part_sh.at[sid])
    plsc.subcore_barrier()
    pltpu.sync_copy(part_sh.at[:, pl.ds(nbase, NS)], red_v)

    fE = jnp.float32(E)

    one = jnp.float32(1.0)

    @plsc.parallel_loop(0, NS // 16, 1, unroll=4)
    def red_body(j):
        b = j * 16
        v = red_v[0, pl.ds(b, 16)]
        for t in range(1, NT):
            v = v + red_v[t, pl.ds(b, 16)]
        acc_v[pl.ds(b, 16)] = one / (v + fE)

    pltpu.sync_copy(acc_v.at[pl.ds(0, NS)], red_sh.at[pl.ds(nbase, NS)])
    plsc.subcore_barrier()

    pltpu.sync_copy(red_sh, acc_v)

    @plsc.parallel_loop(0, NP // 16, 1, unroll=8)
    def zero2_body(i):
        s_v[pl.ds(i * 16, 16)] = zeros16

    half = cid * NV2 * 16

    @plsc.parallel_loop(0, NV2, 1, unroll=8)
    def p2_body(i):
        b = half + i * 16
        r = row_v[pl.ds(b, 16)]
        c = col_v[pl.ds(b, 16)]
        an = ea_v[pl.ds(b, 16)] * plsc.load_gather(acc_v, [r])
        plsc.addupdate_scatter(s_v, [c], an)

    pltpu.sync_copy(s_v, part_sh.at[sid])
    plsc.subcore_barrier()
    pltpu.sync_copy(part_sh.at[:, pl.ds(nbase, NS)], red_v)

    @plsc.parallel_loop(0, NS // 16, 1, unroll=4)
    def red2_body(j):
        b = j * 16
        v = red_v[0, pl.ds(b, 16)]
        for t in range(1, NT):
            v = v + red_v[t, pl.ds(b, 16)]
        acc_v[pl.ds(b, 16)] = v

    pltpu.sync_copy(acc_v.at[pl.ds(0, NS)],
                    s_out_hbm.at[pl.ds(cid * NP + nbase, NS)])


def _edge_kernel(sd, ei):
    mesh = plsc.VectorSubcoreMesh(core_axis_name="c", subcore_axis_name="s",
                                  num_cores=NC)
    return pl.kernel(
        _edge_body,
        mesh=mesh,
        compiler_params=pltpu.CompilerParams(needs_layout_passes=False),
        out_type=jax.ShapeDtypeStruct((NC * NP,), jnp.float32),
        scratch_types=[
            pltpu.VMEM((NP,), jnp.float32),
            pltpu.VMEM((NP,), jnp.float32),
            pltpu.VMEM((EW,), jnp.int32),
            pltpu.VMEM((EW,), jnp.int32),
            pltpu.VMEM((EW,), jnp.float32),
            pltpu.VMEM((NP,), jnp.float32),
            pltpu.VMEM((NT, NS), jnp.float32),
            pltpu.VMEM_SHARED((NT, NP), jnp.float32),
            pltpu.VMEM_SHARED((NP,), jnp.float32),
            pltpu.SemaphoreType.DMA,
        ],
    )(sd, ei)


def _scale_body(xp_ref, s_ref, b_ref, out_ref):
    ir = lax.broadcasted_iota(jnp.int32, (128, 128), 0)
    ic = lax.broadcasted_iota(jnp.int32, (128, 128), 1)
    eye = (ir == ic)
    s_blk = s_ref[0] + s_ref[1]
    for g in range(BLK // 128):
        srow = s_blk[g:g + 1, :]
        dg = jnp.where(eye, jnp.broadcast_to(srow, (128, 128)), 0.0)
        xp_g = xp_ref[pl.ds(g * 128, 128), :]
        out_ref[pl.ds(g * 128, 128), :] = (
            lax.dot_general(dg, xp_g, (((1,), (0,)), ((), ())),
                            preferred_element_type=jnp.float32)
            + b_ref[...]
        )


def _scale(xp, s2, bias_row):
    return pl.pallas_call(
        _scale_body,
        grid=(NP // BLK,),
        in_specs=[
            pl.BlockSpec((BLK, C), lambda i: (i, 0)),
            pl.BlockSpec((NC, BLK // 128, 128), lambda i: (0, i, 0)),
            pl.BlockSpec((1, C), lambda i: (0, 0)),
        ],
        out_specs=pl.BlockSpec((BLK, C), lambda i: (i, 0)),
        out_shape=jax.ShapeDtypeStruct((N, C), jnp.float32),
    )(xp, s2, bias_row)


def kernel(x, W, att, bias, edge_index):
    a2 = jnp.stack([att[0, 0, :C], att[0, 0, C:]], axis=1)
    sd = _project_sd(x, W, a2)
    ei = edge_index.reshape(2 * E)
    s2 = _edge_kernel(sd.reshape(NC * NP), ei)
    xp = _matmul_xp(x, W)
    out = _scale(xp, s2.reshape(NC, NR, 128), bias[None, :])
    return out
